# baseline (speedup 1.0000x reference)
import math

import jax
import jax.numpy as jnp
from jax.experimental import pallas as pl
from jax.experimental.pallas import tpu as pltpu

T = 2048
D = 1024
F = 4096
E = 8
K = 2
CAP = int(math.ceil(T * K / E * 1.2))
CP = 616
FB = 1024
NF = F // FB

_NEG = -3.0e38


def _router_body(x_ref, wg_ref, out_ref, oh_ref, excl_ref):
    x = x_ref[...]
    logits = jax.lax.dot_general(x, wg_ref[...], (((1,), (0,)), ((), ())))
    iota = jax.lax.broadcasted_iota(jnp.int32, (T, E), 1)
    m0 = jnp.max(logits, axis=1, keepdims=True)
    a0 = jnp.min(jnp.where(logits == m0, iota, E), axis=1, keepdims=True)
    l1 = jnp.where(iota == a0, _NEG, logits)
    m1 = jnp.max(l1, axis=1, keepdims=True)
    a1 = jnp.min(jnp.where(l1 == m1, iota, E), axis=1, keepdims=True)
    ed = jnp.exp(m1 - m0)
    g0 = 1.0 / (1.0 + ed)
    g1 = 1.0 - g0
    oh_ref[...] = ((iota == a0) | (iota == a1)).astype(jnp.float32)

    chunk = 256
    r = jax.lax.broadcasted_iota(jnp.int32, (chunk, chunk), 0)
    c = jax.lax.broadcasted_iota(jnp.int32, (chunk, chunk), 1)
    ltri = (r > c).astype(jnp.float32)

    def body(i, carry):
        blk = oh_ref[pl.ds(i * chunk, chunk), :]
        excl_ref[pl.ds(i * chunk, chunk), :] = (
            jax.lax.dot_general(ltri, blk, (((1,), (0,)), ((), ()))) + carry
        )
        return carry + jnp.sum(blk, axis=0, keepdims=True)

    jax.lax.fori_loop(0, T // chunk, body, jnp.zeros((1, E), jnp.float32))

    excl = excl_ref[...]
    p0 = jnp.sum(jnp.where(iota == a0, excl, 0.0), axis=1, keepdims=True)
    p1 = jnp.sum(jnp.where(iota == a1, excl, 0.0), axis=1, keepdims=True)
    k0 = (p0 < CAP).astype(jnp.float32)
    k1 = (p1 < CAP).astype(jnp.float32)
    af0 = a0.astype(jnp.float32)
    af1 = a1.astype(jnp.float32)
    q0 = jnp.where(k0 > 0.0, af0 * CP + p0, -1.0)
    q1 = jnp.where(k1 > 0.0, af1 * CP + p1, -1.0)
    z = jnp.zeros((T, 1), jnp.float32)
    out_ref[...] = jnp.concatenate(
        [q0, q1, g0 * k0, g1 * k1, z, z, z, z], axis=1
    )


def _moe_body(ft_ref, f_ref, x_ref, w1_ref, b1_ref, w2_ref, b2_ref,
              y_ref, buf_s, ye_s):
    e = pl.program_id(0)
    f = pl.program_id(1)
    ef = e.astype(jnp.float32)

    @pl.when(f == 0)
    def _dispatch():
        q0 = ft_ref[0:1, :]
        q1 = ft_ref[1:2, :]
        gci = jax.lax.broadcasted_iota(jnp.int32, (CP, T), 0).astype(jnp.float32)
        gci = gci + ef * CP
        m = ((q0 == gci) | (q1 == gci)).astype(jnp.float32)
        buf_s[...] = jax.lax.dot_general(
            m, x_ref[...], (((1,), (0,)), ((), ()))
        )
        ye_s[...] = jnp.broadcast_to(b2_ref[0], (CP, D))

    h = jax.lax.dot_general(buf_s[...], w1_ref[0], (((1,), (0,)), ((), ())))
    h = h + b1_ref[0]
    h3 = h * h * h
    g = 0.5 * h * (1.0 + jnp.tanh(0.7978845608028654 * (h + 0.044715 * h3)))
    ye_s[...] += jax.lax.dot_general(g, w2_ref[0], (((1,), (0,)), ((), ())))

    @pl.when(f == NF - 1)
    def _combine():
        q0 = f_ref[:, 0:1]
        q1 = f_ref[:, 1:2]
        w0 = f_ref[:, 2:3]
        w1 = f_ref[:, 3:4]
        gci = jax.lax.broadcasted_iota(jnp.int32, (T, CP), 1).astype(jnp.float32)
        gci = gci + ef * CP
        gm = w0 * (q0 == gci).astype(jnp.float32)
        gm = gm + w1 * (q1 == gci).astype(jnp.float32)
        part = jax.lax.dot_general(gm, ye_s[...], (((1,), (0,)), ((), ())))

        @pl.when(e == 0)
        def _set():
            y_ref[...] = part

        @pl.when(e > 0)
        def _acc():
            y_ref[...] += part


@jax.jit
def kernel(x, Wg, W1, b1, W2, b2):
    fields = pl.pallas_call(
        _router_body,
        out_shape=jax.ShapeDtypeStruct((T, E), jnp.float32),
        scratch_shapes=[
            pltpu.VMEM((T, E), jnp.float32),
            pltpu.VMEM((T, E), jnp.float32),
        ],
    )(x, Wg)
    ft = fields.T

    y = pl.pallas_call(
        _moe_body,
        grid=(E, NF),
        in_specs=[
            pl.BlockSpec((E, T), lambda e, f: (0, 0)),
            pl.BlockSpec((T, E), lambda e, f: (0, 0)),
            pl.BlockSpec((T, D), lambda e, f: (0, 0)),
            pl.BlockSpec((1, D, FB), lambda e, f: (e, 0, f)),
            pl.BlockSpec((1, 1, FB), lambda e, f: (e, 0, f)),
            pl.BlockSpec((1, FB, D), lambda e, f: (e, f, 0)),
            pl.BlockSpec((1, 1, D), lambda e, f: (e, 0, 0)),
        ],
        out_specs=pl.BlockSpec((T, D), lambda e, f: (0, 0)),
        out_shape=jax.ShapeDtypeStruct((T, D), jnp.float32),
        scratch_shapes=[
            pltpu.VMEM((CP, D), jnp.float32),
            pltpu.VMEM((CP, D), jnp.float32),
        ],
    )(ft, fields, x, W1, b1.reshape(E, 1, F), W2, b2.reshape(E, 1, D))
    return y
